# baseline (speedup 1.0000x reference)
import jax
import jax.numpy as jnp
from jax import lax
from jax.experimental import pallas as pl

B, N, NP, DN, DC, H = 16384, 50, 64, 16, 128, 64
BLK = 16


def _tc_kernel(idx_ref, adj_ref, nodef_ref, cent_ref,
               Wg1_ref, bg1_ref, Wg2_ref, bg2_ref,
               Wm1a_ref, Wm1b_ref, bm1_ref, Wm2_ref, bm2_ref,
               WvT_ref, bv_ref, out_ref):
    adj = adj_ref[...]
    adj_bf = adj.astype(jnp.bfloat16)
    nodef = nodef_ref[...]

    p = jnp.dot(nodef.astype(jnp.bfloat16), Wg1_ref[...].astype(jnp.bfloat16),
                preferred_element_type=jnp.float32)
    p3 = p.astype(jnp.bfloat16).reshape(BLK, NP, H)[:, :N, :]

    m1 = p3.astype(jnp.float32) + jnp.sum(adj_bf.astype(jnp.float32), axis=2)[:, :, None]
    h1 = jax.nn.relu(m1 + bg1_ref[...])

    idx = idx_ref[...]
    iota = lax.broadcasted_iota(jnp.int32, (BLK, N), 1)
    onehot = (iota == idx).astype(jnp.float32)
    a_ego = jnp.sum(onehot[:, :, None] * adj, axis=1)

    m2e = jnp.sum(a_ego[:, :, None] * h1, axis=1)
    h2e = jax.nn.relu(jnp.dot(m2e, Wg2_ref[...],
                              preferred_element_type=jnp.float32) + bg2_ref[...])

    x = jax.nn.relu(jnp.dot(cent_ref[...], Wm1a_ref[...], preferred_element_type=jnp.float32)
                    + jnp.dot(h2e, Wm1b_ref[...], preferred_element_type=jnp.float32)
                    + bm1_ref[...])
    x = jax.nn.relu(jnp.dot(x, Wm2_ref[...], preferred_element_type=jnp.float32)
                    + bm2_ref[...])
    out_ref[...] = jnp.sum(x * WvT_ref[...], axis=1, keepdims=True) + bv_ref[...]


def kernel(cent_obs, node_obs, adj, agent_id,
           W_gnn1, b_gnn1, W_gnn2, b_gnn2,
           W_mlp1, b_mlp1, W_mlp2, b_mlp2,
           W_v, b_v):
    idx = agent_id.astype(jnp.int32).reshape(B, 1)
    nodef = jnp.pad(node_obs, ((0, 0), (0, NP - N), (0, 0))).reshape(B * NP, DN)
    grid = (B // BLK,)

    const = lambda *shape: pl.BlockSpec(shape, lambda i: (0,) * len(shape))
    out = pl.pallas_call(
        _tc_kernel,
        grid=grid,
        in_specs=[
            pl.BlockSpec((BLK, 1), lambda i: (i, 0)),
            pl.BlockSpec((BLK, N, N), lambda i: (i, 0, 0)),
            pl.BlockSpec((BLK * NP, DN), lambda i: (i, 0)),
            pl.BlockSpec((BLK, DC), lambda i: (i, 0)),
            const(DN, H), const(1, H),
            const(H, H), const(1, H),
            const(DC, H), const(H, H), const(1, H),
            const(H, H), const(1, H),
            const(1, H), const(1, 1),
        ],
        out_specs=pl.BlockSpec((BLK, 1), lambda i: (i, 0)),
        out_shape=jax.ShapeDtypeStruct((B, 1), jnp.float32),
    )(idx, adj, nodef, cent_obs,
      W_gnn1, b_gnn1.reshape(1, H), W_gnn2, b_gnn2.reshape(1, H),
      W_mlp1[:DC], W_mlp1[DC:], b_mlp1.reshape(1, H),
      W_mlp2, b_mlp2.reshape(1, H),
      W_v.reshape(1, H), b_v.reshape(1, 1))
    return out


# speedup vs baseline: 1.5762x; 1.5762x over previous
import jax
import jax.numpy as jnp
from jax import lax
from jax.experimental import pallas as pl

B, N, NP, DN, DC, H = 16384, 50, 64, 16, 128, 64
BLK = 64


def _tc_kernel(idx_ref, adj_ref, nodef_ref, cent_ref,
               Wg1_ref, bg1_ref, Wg2_ref, bg2_ref,
               Wm1a_ref, Wm1b_ref, bm1_ref, Wm2_ref, bm2_ref,
               WvT_ref, bv_ref, out_ref):
    adj = adj_ref[...]
    adj_bf = adj.astype(jnp.bfloat16)
    nodef = nodef_ref[...]

    p = jnp.dot(nodef.astype(jnp.bfloat16), Wg1_ref[...].astype(jnp.bfloat16),
                preferred_element_type=jnp.float32)
    p3 = p.astype(jnp.bfloat16).reshape(BLK, NP, H)[:, :N, :]

    m1 = lax.dot_general(adj_bf, p3, (((2,), (1,)), ((0,), (0,))),
                         preferred_element_type=jnp.float32)
    h1 = jax.nn.relu(m1 + bg1_ref[...])

    idx = idx_ref[...]
    iota = lax.broadcasted_iota(jnp.int32, (BLK, N), 1)
    onehot = (iota == idx).astype(jnp.float32).reshape(BLK, 1, N)
    a_ego = lax.dot_general(onehot, adj, (((2,), (1,)), ((0,), (0,))),
                            preferred_element_type=jnp.float32)

    m2e = lax.dot_general(a_ego, h1, (((2,), (1,)), ((0,), (0,))),
                          preferred_element_type=jnp.float32).reshape(BLK, H)
    h2e = jax.nn.relu(jnp.dot(m2e, Wg2_ref[...],
                              preferred_element_type=jnp.float32) + bg2_ref[...])

    x = jax.nn.relu(jnp.dot(cent_ref[...], Wm1a_ref[...], preferred_element_type=jnp.float32)
                    + jnp.dot(h2e, Wm1b_ref[...], preferred_element_type=jnp.float32)
                    + bm1_ref[...])
    x = jax.nn.relu(jnp.dot(x, Wm2_ref[...], preferred_element_type=jnp.float32)
                    + bm2_ref[...])
    out_ref[...] = jnp.sum(x * WvT_ref[...], axis=1, keepdims=True) + bv_ref[...]


def kernel(cent_obs, node_obs, adj, agent_id,
           W_gnn1, b_gnn1, W_gnn2, b_gnn2,
           W_mlp1, b_mlp1, W_mlp2, b_mlp2,
           W_v, b_v):
    idx = agent_id.astype(jnp.int32).reshape(B, 1)
    nodef = jnp.pad(node_obs, ((0, 0), (0, NP - N), (0, 0))).reshape(B * NP, DN)
    grid = (B // BLK,)

    const = lambda *shape: pl.BlockSpec(shape, lambda i: (0,) * len(shape))
    out = pl.pallas_call(
        _tc_kernel,
        grid=grid,
        in_specs=[
            pl.BlockSpec((BLK, 1), lambda i: (i, 0)),
            pl.BlockSpec((BLK, N, N), lambda i: (i, 0, 0)),
            pl.BlockSpec((BLK * NP, DN), lambda i: (i, 0)),
            pl.BlockSpec((BLK, DC), lambda i: (i, 0)),
            const(DN, H), const(1, H),
            const(H, H), const(1, H),
            const(DC, H), const(H, H), const(1, H),
            const(H, H), const(1, H),
            const(1, H), const(1, 1),
        ],
        out_specs=pl.BlockSpec((BLK, 1), lambda i: (i, 0)),
        out_shape=jax.ShapeDtypeStruct((B, 1), jnp.float32),
    )(idx, adj, nodef, cent_obs,
      W_gnn1, b_gnn1.reshape(1, H), W_gnn2, b_gnn2.reshape(1, H),
      W_mlp1[:DC], W_mlp1[DC:], b_mlp1.reshape(1, H),
      W_mlp2, b_mlp2.reshape(1, H),
      W_v.reshape(1, H), b_v.reshape(1, 1))
    return out


# speedup vs baseline: 1.6969x; 1.0766x over previous
import jax
import jax.numpy as jnp
from jax import lax
from jax.experimental import pallas as pl

B, N, NP, DN, DC, H = 16384, 50, 64, 16, 128, 64
BLK = 128


def _tc_kernel(idx_ref, adj_ref, nodef_ref, cent_ref,
               Wg1_ref, bg1_ref, Wg2_ref, bg2_ref,
               Wm1a_ref, Wm1b_ref, bm1_ref, Wm2_ref, bm2_ref,
               WvT_ref, bv_ref, out_ref):
    adj = adj_ref[...]
    nodef = nodef_ref[...]

    p = jnp.dot(nodef, Wg1_ref[...],
                preferred_element_type=jnp.float32)
    p3 = p.reshape(BLK, NP, H)[:, :N, :]

    m1 = lax.dot_general(adj, p3, (((2,), (1,)), ((0,), (0,))),
                         preferred_element_type=jnp.float32)
    h1 = jax.nn.relu(m1 + bg1_ref[...])

    idx = idx_ref[...]
    iota = lax.broadcasted_iota(jnp.int32, (BLK, N), 1)
    onehot = (iota == idx).astype(jnp.float32).reshape(BLK, 1, N)
    a_ego = lax.dot_general(onehot, adj, (((2,), (1,)), ((0,), (0,))),
                            preferred_element_type=jnp.float32)

    m2e = lax.dot_general(a_ego, h1, (((2,), (1,)), ((0,), (0,))),
                          preferred_element_type=jnp.float32).reshape(BLK, H)
    h2e = jax.nn.relu(jnp.dot(m2e, Wg2_ref[...],
                              preferred_element_type=jnp.float32) + bg2_ref[...])

    x = jax.nn.relu(jnp.dot(cent_ref[...], Wm1a_ref[...], preferred_element_type=jnp.float32)
                    + jnp.dot(h2e, Wm1b_ref[...], preferred_element_type=jnp.float32)
                    + bm1_ref[...])
    x = jax.nn.relu(jnp.dot(x, Wm2_ref[...], preferred_element_type=jnp.float32)
                    + bm2_ref[...])
    out_ref[...] = jnp.sum(x * WvT_ref[...], axis=1, keepdims=True) + bv_ref[...]


def kernel(cent_obs, node_obs, adj, agent_id,
           W_gnn1, b_gnn1, W_gnn2, b_gnn2,
           W_mlp1, b_mlp1, W_mlp2, b_mlp2,
           W_v, b_v):
    idx = agent_id.astype(jnp.int32).reshape(B, 1)
    nodef = jnp.pad(node_obs, ((0, 0), (0, NP - N), (0, 0))).reshape(B * NP, DN)
    grid = (B // BLK,)

    const = lambda *shape: pl.BlockSpec(shape, lambda i: (0,) * len(shape))
    out = pl.pallas_call(
        _tc_kernel,
        grid=grid,
        in_specs=[
            pl.BlockSpec((BLK, 1), lambda i: (i, 0)),
            pl.BlockSpec((BLK, N, N), lambda i: (i, 0, 0)),
            pl.BlockSpec((BLK * NP, DN), lambda i: (i, 0)),
            pl.BlockSpec((BLK, DC), lambda i: (i, 0)),
            const(DN, H), const(1, H),
            const(H, H), const(1, H),
            const(DC, H), const(H, H), const(1, H),
            const(H, H), const(1, H),
            const(1, H), const(1, 1),
        ],
        out_specs=pl.BlockSpec((BLK, 1), lambda i: (i, 0)),
        out_shape=jax.ShapeDtypeStruct((B, 1), jnp.float32),
    )(idx, adj, nodef, cent_obs,
      W_gnn1, b_gnn1.reshape(1, H), W_gnn2, b_gnn2.reshape(1, H),
      W_mlp1[:DC], W_mlp1[DC:], b_mlp1.reshape(1, H),
      W_mlp2, b_mlp2.reshape(1, H),
      W_v.reshape(1, H), b_v.reshape(1, 1))
    return out


# speedup vs baseline: 1.7768x; 1.0471x over previous
import jax
import jax.numpy as jnp
from jax import lax
from jax.experimental import pallas as pl

B, N, NP, DN, DC, H = 16384, 50, 64, 16, 128, 64
BLK = 256


def _tc_kernel(idx_ref, adj_ref, nodef_ref, cent_ref,
               Wg1_ref, bg1_ref, Wg2_ref, bg2_ref,
               Wm1a_ref, Wm1b_ref, bm1_ref, Wm2_ref, bm2_ref,
               WvT_ref, bv_ref, out_ref):
    adj = adj_ref[...]
    nodef = nodef_ref[...]

    p = jnp.dot(nodef, Wg1_ref[...],
                preferred_element_type=jnp.float32)
    p3 = p.reshape(BLK, NP, H)[:, :N, :]

    m1 = lax.dot_general(adj, p3, (((2,), (1,)), ((0,), (0,))),
                         preferred_element_type=jnp.float32)
    h1 = jax.nn.relu(m1 + bg1_ref[...])

    idx = idx_ref[...]
    iota = lax.broadcasted_iota(jnp.int32, (BLK, N), 1)
    onehot = (iota == idx).astype(jnp.float32).reshape(BLK, 1, N)
    a_ego = lax.dot_general(onehot, adj, (((2,), (1,)), ((0,), (0,))),
                            preferred_element_type=jnp.float32)

    m2e = lax.dot_general(a_ego, h1, (((2,), (1,)), ((0,), (0,))),
                          preferred_element_type=jnp.float32).reshape(BLK, H)
    h2e = jax.nn.relu(jnp.dot(m2e, Wg2_ref[...],
                              preferred_element_type=jnp.float32) + bg2_ref[...])

    x = jax.nn.relu(jnp.dot(cent_ref[...], Wm1a_ref[...], preferred_element_type=jnp.float32)
                    + jnp.dot(h2e, Wm1b_ref[...], preferred_element_type=jnp.float32)
                    + bm1_ref[...])
    x = jax.nn.relu(jnp.dot(x, Wm2_ref[...], preferred_element_type=jnp.float32)
                    + bm2_ref[...])
    out_ref[...] = jnp.sum(x * WvT_ref[...], axis=1, keepdims=True) + bv_ref[...]


def kernel(cent_obs, node_obs, adj, agent_id,
           W_gnn1, b_gnn1, W_gnn2, b_gnn2,
           W_mlp1, b_mlp1, W_mlp2, b_mlp2,
           W_v, b_v):
    idx = agent_id.astype(jnp.int32).reshape(B, 1)
    nodef = jnp.pad(node_obs, ((0, 0), (0, NP - N), (0, 0))).reshape(B * NP, DN)
    grid = (B // BLK,)

    const = lambda *shape: pl.BlockSpec(shape, lambda i: (0,) * len(shape))
    out = pl.pallas_call(
        _tc_kernel,
        grid=grid,
        in_specs=[
            pl.BlockSpec((BLK, 1), lambda i: (i, 0)),
            pl.BlockSpec((BLK, N, N), lambda i: (i, 0, 0)),
            pl.BlockSpec((BLK * NP, DN), lambda i: (i, 0)),
            pl.BlockSpec((BLK, DC), lambda i: (i, 0)),
            const(DN, H), const(1, H),
            const(H, H), const(1, H),
            const(DC, H), const(H, H), const(1, H),
            const(H, H), const(1, H),
            const(1, H), const(1, 1),
        ],
        out_specs=pl.BlockSpec((BLK, 1), lambda i: (i, 0)),
        out_shape=jax.ShapeDtypeStruct((B, 1), jnp.float32),
    )(idx, adj, nodef, cent_obs,
      W_gnn1, b_gnn1.reshape(1, H), W_gnn2, b_gnn2.reshape(1, H),
      W_mlp1[:DC], W_mlp1[DC:], b_mlp1.reshape(1, H),
      W_mlp2, b_mlp2.reshape(1, H),
      W_v.reshape(1, H), b_v.reshape(1, 1))
    return out
